# Initial kernel scaffold; baseline (speedup 1.0000x reference)
import functools

import jax
import jax.numpy as jnp
from jax import lax
from jax.experimental import pallas as pl
from jax.experimental.pallas import tpu as pltpu
from jax.experimental.pallas import tpu_sc as plsc

VOCAB = 1000000
DIM = 64
SEQ = 50
BATCH = 16384

NC = 2
NS = 16
NW = NC * NS
LANES = 16

SEQ_PER_W = BATCH // NW
CHUNK_SEQ = 16
CHUNK_ROWS = CHUNK_SEQ * SEQ
N_CHUNKS = SEQ_PER_W // CHUNK_SEQ
DCOLS = DIM // LANES


def _emb_kernel(x_hbm, tab_hbm, pos_hbm, out_hbm, idx_v, rows_v, pos_v, sem):
    wid = lax.axis_index("s") * NC + lax.axis_index("c")
    base_w = wid * (SEQ_PER_W * SEQ)

    pltpu.sync_copy(pos_hbm, pos_v)

    def chunk_body(k, _):
        base = base_w + k * CHUNK_ROWS
        pltpu.sync_copy(x_hbm.at[pl.ds(base, CHUNK_ROWS)], idx_v)
        pltpu.async_copy(tab_hbm.at[idx_v], rows_v, sem).wait()

        def s_body(s, _):
            for dcol in range(DCOLS):
                dofs = dcol * LANES
                pvec = pos_v[s, pl.ds(dofs, LANES)]
                for c in range(CHUNK_SEQ):
                    row = c * SEQ + s
                    rows_v[row, pl.ds(dofs, LANES)] = (
                        rows_v[row, pl.ds(dofs, LANES)] + pvec)
            return 0

        lax.fori_loop(0, SEQ, s_body, 0)
        pltpu.sync_copy(rows_v, out_hbm.at[pl.ds(base, CHUNK_ROWS)])
        return 0

    lax.fori_loop(0, N_CHUNKS, chunk_body, 0)


def kernel(x, token_table, pos_table):
    x_flat = x.reshape(-1).astype(jnp.int32)
    mesh = plsc.VectorSubcoreMesh(core_axis_name="c", subcore_axis_name="s")
    k = pl.kernel(
        _emb_kernel,
        mesh=mesh,
        out_type=jax.ShapeDtypeStruct((BATCH * SEQ, DIM), jnp.float32),
        scratch_types=[
            pltpu.VMEM((CHUNK_ROWS,), jnp.int32),
            pltpu.VMEM((CHUNK_ROWS, DIM), jnp.float32),
            pltpu.VMEM((SEQ, DIM), jnp.float32),
            pltpu.SemaphoreType.DMA,
        ],
    )
    out = k(x_flat, token_table, pos_table)
    return out.reshape(BATCH, SEQ, DIM)


# speedup vs baseline: 1.7453x; 1.7453x over previous
import functools

import jax
import jax.numpy as jnp
from jax import lax
from jax.experimental import pallas as pl
from jax.experimental.pallas import tpu as pltpu
from jax.experimental.pallas import tpu_sc as plsc

VOCAB = 1000000
DIM = 64
SEQ = 50
BATCH = 16384

NC = 2
NS = 16
NW = NC * NS
LANES = 16

SEQ_PER_W = BATCH // NW
CHUNK_SEQ = 16
CHUNK_ROWS = CHUNK_SEQ * SEQ
N_CHUNKS = SEQ_PER_W // CHUNK_SEQ
DCOLS = DIM // LANES


def _emb_kernel(x_hbm, tab_hbm, pos_hbm, out_hbm, idx_v, rows_v, pos_v, sem):
    wid = lax.axis_index("s") * NC + lax.axis_index("c")
    base_w = wid * (SEQ_PER_W * SEQ)

    pltpu.sync_copy(pos_hbm, pos_v)

    def chunk_body(k, _):
        base = base_w + k * CHUNK_ROWS
        pltpu.sync_copy(x_hbm.at[pl.ds(base, CHUNK_ROWS)], idx_v)
        pltpu.async_copy(tab_hbm.at[idx_v], rows_v, sem).wait()

        def s_body(s, _):
            for dcol in range(DCOLS):
                dofs = dcol * LANES
                pvec = pos_v[s, pl.ds(dofs, LANES)]
                for c in range(CHUNK_SEQ):
                    row = c * SEQ + s
                    rows_v[row, pl.ds(dofs, LANES)] = (
                        rows_v[row, pl.ds(dofs, LANES)] + pvec)
            return 0

        lax.fori_loop(0, SEQ, s_body, 0)
        pltpu.sync_copy(rows_v, out_hbm.at[pl.ds(base, CHUNK_ROWS)])
        return 0

    lax.fori_loop(0, N_CHUNKS, chunk_body, 0)


def kernel(x, token_table, pos_table):
    x_flat = x.reshape(-1).astype(jnp.int32)
    mesh = plsc.VectorSubcoreMesh(core_axis_name="c", subcore_axis_name="s")
    k = pl.kernel(
        _emb_kernel,
        mesh=mesh,
        compiler_params=pltpu.CompilerParams(use_tc_tiling_on_sc=False),
        out_type=jax.ShapeDtypeStruct((BATCH * SEQ, DIM), jnp.float32),
        scratch_types=[
            pltpu.VMEM((CHUNK_ROWS,), jnp.int32),
            pltpu.VMEM((CHUNK_ROWS, DIM), jnp.float32),
            pltpu.VMEM((SEQ, DIM), jnp.float32),
            pltpu.SemaphoreType.DMA,
        ],
    )
    out = k(x_flat, token_table, pos_table)
    return out.reshape(BATCH, SEQ, DIM)


# speedup vs baseline: 2.2600x; 1.2949x over previous
import jax
import jax.numpy as jnp
from jax import lax
from jax.experimental import pallas as pl
from jax.experimental.pallas import tpu as pltpu
from jax.experimental.pallas import tpu_sc as plsc

VOCAB = 1000000
DIM = 64
SEQ = 50
BATCH = 16384

NC = 2
NS = 16
NW = NC * NS
LANES = 16

SEQ_PER_W = BATCH // NW
CHUNK_SEQ = 16
CHUNK_ROWS = CHUNK_SEQ * SEQ
N_CHUNKS = SEQ_PER_W // CHUNK_SEQ
DCOLS = DIM // LANES

FMT_BV = 4096
HALF_BLOCKS = 123
HALF = HALF_BLOCKS * FMT_BV


def _fmt_kernel(a_ref, b_ref, p_ref):
    p_ref[...] = jnp.concatenate(
        [jnp.transpose(a_ref[...]), jnp.transpose(b_ref[...])], axis=1)


def _table_rowmajor(table_t):
    p = pl.pallas_call(
        _fmt_kernel,
        grid=(HALF_BLOCKS,),
        in_specs=[
            pl.BlockSpec((DIM, FMT_BV), lambda i: (0, i)),
            pl.BlockSpec((DIM, FMT_BV),
                         lambda i: (0, jnp.minimum(i + HALF_BLOCKS,
                                                   VOCAB // FMT_BV))),
        ],
        out_specs=pl.BlockSpec((FMT_BV, 2 * DIM), lambda i: (i, 0)),
        out_shape=jax.ShapeDtypeStruct((HALF, 2 * DIM), jnp.float32),
    )(table_t, table_t)
    return p.reshape(2 * HALF, DIM)


def _emb_kernel(x_hbm, tab_hbm, pos_hbm, out_hbm, idx_v, rows_v, pos_v, sem):
    wid = lax.axis_index("s") * NC + lax.axis_index("c")
    base_w = wid * (SEQ_PER_W * SEQ)

    pltpu.sync_copy(pos_hbm, pos_v)

    def chunk_body(k, _):
        base = base_w + k * CHUNK_ROWS
        pltpu.sync_copy(x_hbm.at[pl.ds(base, CHUNK_ROWS)], idx_v)
        pltpu.async_copy(tab_hbm.at[idx_v], rows_v, sem).wait()

        def s_body(s, _):
            for dcol in range(DCOLS):
                dofs = dcol * LANES
                pvec = pos_v[s, pl.ds(dofs, LANES)]
                for c in range(CHUNK_SEQ):
                    row = c * SEQ + s
                    rows_v[row, pl.ds(dofs, LANES)] = (
                        rows_v[row, pl.ds(dofs, LANES)] + pvec)
            return 0

        lax.fori_loop(0, SEQ, s_body, 0)
        pltpu.sync_copy(rows_v, out_hbm.at[pl.ds(base, CHUNK_ROWS)])
        return 0

    lax.fori_loop(0, N_CHUNKS, chunk_body, 0)


def kernel(x, token_table, pos_table):
    x_flat = x.reshape(-1).astype(jnp.int32)
    x_flat = jnp.where(x_flat >= HALF, 2 * x_flat - (2 * HALF - 1), 2 * x_flat)
    tab = _table_rowmajor(token_table.T)
    mesh = plsc.VectorSubcoreMesh(core_axis_name="c", subcore_axis_name="s")
    k = pl.kernel(
        _emb_kernel,
        mesh=mesh,
        compiler_params=pltpu.CompilerParams(use_tc_tiling_on_sc=False),
        out_type=jax.ShapeDtypeStruct((BATCH * SEQ, DIM), jnp.float32),
        scratch_types=[
            pltpu.VMEM((CHUNK_ROWS,), jnp.int32),
            pltpu.VMEM((CHUNK_ROWS, DIM), jnp.float32),
            pltpu.VMEM((SEQ, DIM), jnp.float32),
            pltpu.SemaphoreType.DMA,
        ],
    )
    out = k(x_flat, tab, pos_table)
    return out.reshape(BATCH, SEQ, DIM)
